# Initial kernel scaffold; baseline (speedup 1.0000x reference)
import jax
import jax.numpy as jnp
from jax.experimental import pallas as pl


def kernel(user_idx, item_idx, item_attr1_idx, item_attr2_idx, edge_index, user_table, item_table, attr1_table, attr2_table, W1, b1, W2, b2, W3, b3):
    raise NotImplementedError("write your pallas kernel here")


# speedup vs baseline: 18.1230x; 18.1230x over previous
import functools

import jax
import jax.numpy as jnp
from jax import lax
from jax.experimental import pallas as pl
from jax.experimental.pallas import tpu as pltpu
from jax.experimental.pallas import tpu_sc as plsc

NN = 100000
NP = 100352
EE = 1600000
D = 32
HD = 16
NC, NS = 2, 16
RPW = NP // (NC * NS)
EPD = EE // (NC * NS)
EPS = EE // NS
RPS = NP // NS
CH = 800
FP = 80128
FPW = FP // (NC * NS)

_mesh = plsc.VectorSubcoreMesh(core_axis_name="c", subcore_axis_name="s")
f32 = jnp.float32
i32 = jnp.int32


def _worker_id():
  return lax.axis_index("s") * NC + lax.axis_index("c")


@jax.jit
def _embed_gather(table_cat, idx_all):
  @functools.partial(
      pl.kernel,
      out_type=jax.ShapeDtypeStruct((NP, D), f32),
      mesh=_mesh,
      scratch_types=[pltpu.VMEM((RPW,), i32), pltpu.VMEM((RPW, D), f32)],
      compiler_params=pltpu.CompilerParams(use_tc_tiling_on_sc=False),
  )
  def k(tab_hbm, idx_hbm, out_hbm, idx_v, rows_v):
    base = _worker_id() * RPW
    pltpu.sync_copy(idx_hbm.at[pl.ds(base, RPW)], idx_v)
    pltpu.sync_copy(tab_hbm.at[idx_v], rows_v)
    pltpu.sync_copy(rows_v, out_hbm.at[pl.ds(base, RPW)])

  return k(table_cat, idx_all)


@jax.jit
def _final_gather(s_nodes, fidx):
  @functools.partial(
      pl.kernel,
      out_type=jax.ShapeDtypeStruct((FP, D), f32),
      mesh=_mesh,
      scratch_types=[pltpu.VMEM((FPW,), i32), pltpu.VMEM((FPW, D), f32)],
      compiler_params=pltpu.CompilerParams(use_tc_tiling_on_sc=False),
  )
  def k(src_hbm, idx_hbm, out_hbm, idx_v, rows_v):
    base = _worker_id() * FPW
    pltpu.sync_copy(idx_hbm.at[pl.ds(base, FPW)], idx_v)
    pltpu.sync_copy(src_hbm.at[idx_v], rows_v)
    pltpu.sync_copy(rows_v, out_hbm.at[pl.ds(base, FPW)])

  return k(s_nodes, fidx)


DCH = 1000


DCH = 1000


@jax.jit
def _degree(dsts, zrows):
  @functools.partial(
      pl.kernel,
      out_type=jax.ShapeDtypeStruct((NC, NP, HD), f32),
      mesh=_mesh,
      scratch_types=[
          pltpu.VMEM((DCH, HD), f32),
          pltpu.VMEM((DCH,), i32),
          pltpu.VMEM_SHARED((NP, HD), f32),
      ],
      compiler_params=pltpu.CompilerParams(use_tc_tiling_on_sc=False),
  )
  def k(dst_hbm, zero_hbm, out_hbm, ones_v, dst_v, deg_sh):
    c = lax.axis_index("c")
    s = lax.axis_index("s")
    w = _worker_id()
    ones16 = jnp.ones((HD,), f32)

    @pl.loop(0, DCH)
    def _(i):
      ones_v[i] = ones16

    pltpu.sync_copy(zero_hbm.at[pl.ds(s * RPS, RPS)],
                    deg_sh.at[pl.ds(s * RPS, RPS)])
    plsc.subcore_barrier()

    @pl.loop(0, EPD // DCH)
    def _(kk):
      base = w * EPD + kk * DCH
      pltpu.sync_copy(dst_hbm.at[pl.ds(base, DCH)], dst_v)
      pltpu.sync_copy(ones_v, deg_sh.at[dst_v], add=True)

    plsc.subcore_barrier()

    def out(oh):
      pltpu.sync_copy(deg_sh.at[pl.ds(s * RPS, RPS)],
                      oh.at[pl.ds(s * RPS, RPS)])

    @pl.when(c == 0)
    def _():
      out(out_hbm.at[0])

    @pl.when(c == 1)
    def _():
      out(out_hbm.at[1])

  return k(dsts, zrows)


@jax.jit
def _propagate(z2, srcs, dsts, zrows):
  @functools.partial(
      pl.kernel,
      out_type=jax.ShapeDtypeStruct((2, NP, HD), f32),
      mesh=_mesh,
      scratch_types=[
          pltpu.VMEM((CH,), i32),
          pltpu.VMEM((CH,), i32),
          pltpu.VMEM((CH, HD), f32),
          pltpu.VMEM_SHARED((NP, HD), f32),
      ],
      compiler_params=pltpu.CompilerParams(use_tc_tiling_on_sc=False),
  )
  def k(z_hbm, src_hbm, dst_hbm, zero_hbm, out_hbm, src_v, dst_v, rows_v,
        acc_sh):
    c = lax.axis_index("c")
    s = lax.axis_index("s")
    rbase = s * RPS
    pltpu.sync_copy(zero_hbm.at[pl.ds(rbase, RPS)],
                    acc_sh.at[pl.ds(rbase, RPS)])
    plsc.subcore_barrier()

    def run(zh, oh):
      @pl.loop(0, EPS // CH)
      def _(kk):
        base = s * EPS + kk * CH
        pltpu.sync_copy(src_hbm.at[pl.ds(base, CH)], src_v)
        pltpu.sync_copy(dst_hbm.at[pl.ds(base, CH)], dst_v)
        pltpu.sync_copy(zh.at[src_v], rows_v)
        pltpu.sync_copy(rows_v, acc_sh.at[dst_v], add=True)

      plsc.subcore_barrier()
      pltpu.sync_copy(acc_sh.at[pl.ds(rbase, RPS)], oh.at[pl.ds(rbase, RPS)])

    @pl.when(c == 0)
    def _():
      run(z_hbm.at[0], out_hbm.at[0])

    @pl.when(c == 1)
    def _():
      run(z_hbm.at[1], out_hbm.at[1])

  return k(z2, srcs, dsts, zrows)


_BT = 2048
_BR = 1024
_BS = 1000


@jax.jit
def _tc_deg_reduce(degp):
  def body(degp_ref, out_ref):
    d = degp_ref[0][:, 0:1] + degp_ref[1][:, 0:1] + 1.0
    r = 1.0 / jnp.sqrt(d)
    out_ref[...] = jnp.concatenate([r, 1.0 / d], axis=1)

  return pl.pallas_call(
      body,
      out_shape=jax.ShapeDtypeStruct((NP, 2), f32),
      grid=(NP // _BR,),
      in_specs=[pl.BlockSpec((NC, _BR, HD), lambda i: (0, i, 0))],
      out_specs=pl.BlockSpec((_BR, 2), lambda i: (i, 0)),
  )(degp)


@jax.jit
def _tc_pre(x0, disT, W1):
  def body(x_ref, d_ref, w_ref, xw_ref, z_ref):
    xw = jnp.dot(x_ref[...].astype(jnp.bfloat16),
                 w_ref[...].astype(jnp.bfloat16), preferred_element_type=f32)
    xw_ref[...] = xw
    z = xw * d_ref[:, 0:1]
    z_ref[0] = z[:, :HD]
    z_ref[1] = z[:, HD:]

  return pl.pallas_call(
      body,
      out_shape=(
          jax.ShapeDtypeStruct((NP, D), f32),
          jax.ShapeDtypeStruct((2, NP, HD), f32),
      ),
      grid=(NP // _BR,),
      in_specs=[
          pl.BlockSpec((_BR, D), lambda i: (i, 0)),
          pl.BlockSpec((_BR, 2), lambda i: (i, 0)),
          pl.BlockSpec((D, D), lambda i: (0, 0)),
      ],
      out_specs=(
          pl.BlockSpec((_BR, D), lambda i: (i, 0)),
          pl.BlockSpec((2, _BR, HD), lambda i: (0, i, 0)),
      ),
  )(x0, disT, W1)


@jax.jit
def _tc_post(acc2, xw, disT, s_sum, brow, w_next):
  def body(a_ref, xw_ref, d_ref, s_ref, b_ref, w_ref,
           so_ref, xwn_ref, zn_ref):
    acc = jnp.concatenate([a_ref[0], a_ref[1]], axis=1)
    dis = d_ref[:, 0:1]
    inv = d_ref[:, 1:2]
    xl = dis * acc + inv * xw_ref[...] + b_ref[...]
    so_ref[...] = s_ref[...] + xl
    xwn = jnp.dot(xl.astype(jnp.bfloat16), w_ref[...].astype(jnp.bfloat16),
                  preferred_element_type=f32)
    xwn_ref[...] = xwn
    z = xwn * dis
    zn_ref[0] = z[:, :HD]
    zn_ref[1] = z[:, HD:]

  return pl.pallas_call(
      body,
      out_shape=(
          jax.ShapeDtypeStruct((NP, D), f32),
          jax.ShapeDtypeStruct((NP, D), f32),
          jax.ShapeDtypeStruct((2, NP, HD), f32),
      ),
      grid=(NP // _BR,),
      in_specs=[
          pl.BlockSpec((2, _BR, HD), lambda i: (0, i, 0)),
          pl.BlockSpec((_BR, D), lambda i: (i, 0)),
          pl.BlockSpec((_BR, 2), lambda i: (i, 0)),
          pl.BlockSpec((_BR, D), lambda i: (i, 0)),
          pl.BlockSpec((1, D), lambda i: (0, 0)),
          pl.BlockSpec((D, D), lambda i: (0, 0)),
      ],
      out_specs=(
          pl.BlockSpec((_BR, D), lambda i: (i, 0)),
          pl.BlockSpec((_BR, D), lambda i: (i, 0)),
          pl.BlockSpec((2, _BR, HD), lambda i: (0, i, 0)),
      ),
  )(acc2, xw, disT, s_sum, brow, w_next)


@jax.jit
def _tc_score(rows):
  def body(u_ref, v_ref, o_ref):
    o_ref[...] = jnp.sum(u_ref[...] * v_ref[...], axis=1,
                         keepdims=True) * (1.0 / 16.0)

  nu = 40000
  return pl.pallas_call(
      body,
      out_shape=jax.ShapeDtypeStruct((nu, 1), f32),
      grid=(nu // _BS,),
      in_specs=[
          pl.BlockSpec((_BS, D), lambda i: (i, 0)),
          pl.BlockSpec((_BS, D), lambda i: (i + nu // _BS, 0)),
      ],
      out_specs=pl.BlockSpec((_BS, 1), lambda i: (i, 0)),
  )(rows, rows)


def kernel(user_idx, item_idx, item_attr1_idx, item_attr2_idx, edge_index,
           user_table, item_table, attr1_table, attr2_table,
           W1, b1, W2, b2, W3, b3):
  user_idx = user_idx.astype(i32)
  item_idx = item_idx.astype(i32)
  nu = user_idx.shape[0]

  table_cat = jnp.concatenate(
      [user_table, item_table, attr1_table, attr2_table], axis=0)
  idx_all = jnp.concatenate([
      user_idx,
      item_idx + user_table.shape[0],
      item_attr1_idx.astype(i32) + (user_table.shape[0] + item_table.shape[0]),
      item_attr2_idx.astype(i32)
      + (user_table.shape[0] + item_table.shape[0] + attr1_table.shape[0]),
      jnp.zeros((NP - NN,), i32),
  ])
  srcs = edge_index[0].astype(i32)
  dsts = edge_index[1].astype(i32)
  zrows = jnp.zeros((NP, HD), f32)

  x0 = _embed_gather(table_cat, idx_all)
  degp = _degree(dsts, zrows)
  disT = _tc_deg_reduce(degp)

  xw, z = _tc_pre(x0, disT, W1)
  s_sum = x0
  for b, w_next in ((b1, W2), (b2, W3), (b3, W3)):
    acc2 = _propagate(z, srcs, dsts, zrows)
    s_sum, xw, z = _tc_post(acc2, xw, disT, s_sum, b.reshape(1, D), w_next)

  fidx = jnp.concatenate(
      [user_idx, item_idx + nu, jnp.zeros((FP - 2 * nu,), i32)])
  rows = _final_gather(s_sum, fidx)
  score = _tc_score(rows)
  return score.reshape(nu)


# speedup vs baseline: 29.8410x; 1.6466x over previous
import functools

import jax
import jax.numpy as jnp
from jax import lax
from jax.experimental import pallas as pl
from jax.experimental.pallas import tpu as pltpu
from jax.experimental.pallas import tpu_sc as plsc

NN = 100000
NP = 100352
EE = 1600000
D = 32
HD = 16
NC, NS = 2, 16
RPW = NP // (NC * NS)
EPD = EE // (NC * NS)
EPS = EE // NS
RPS = NP // NS
CH = 800
WQ = RPS // 8
FP = 80128
FPW = FP // (NC * NS)
NW = NP // 4

_mesh = plsc.VectorSubcoreMesh(core_axis_name="c", subcore_axis_name="s")
_sc_params = pltpu.CompilerParams(use_tc_tiling_on_sc=False)
f32 = jnp.float32
i32 = jnp.int32


def _worker_id():
  return lax.axis_index("s") * NC + lax.axis_index("c")


@jax.jit
def _embed_gather(table_cat, idx_all):
  @functools.partial(
      pl.kernel,
      out_type=jax.ShapeDtypeStruct((NP, D), f32),
      mesh=_mesh,
      scratch_types=[pltpu.VMEM((RPW,), i32), pltpu.VMEM((RPW, D), f32)],
      compiler_params=_sc_params,
  )
  def k(tab_hbm, idx_hbm, out_hbm, idx_v, rows_v):
    base = _worker_id() * RPW
    pltpu.sync_copy(idx_hbm.at[pl.ds(base, RPW)], idx_v)
    pltpu.sync_copy(tab_hbm.at[idx_v], rows_v)
    pltpu.sync_copy(rows_v, out_hbm.at[pl.ds(base, RPW)])

  return k(table_cat, idx_all)


@jax.jit
def _final_gather(s_wide, fidx):
  @functools.partial(
      pl.kernel,
      out_type=jax.ShapeDtypeStruct((FP, D), f32),
      mesh=_mesh,
      scratch_types=[pltpu.VMEM((FPW,), i32), pltpu.VMEM((FPW, D), f32)],
      compiler_params=_sc_params,
  )
  def k(src_hbm, idx_hbm, out_hbm, idx_v, rows_v):
    base = _worker_id() * FPW
    pltpu.sync_copy(idx_hbm.at[pl.ds(base, FPW)], idx_v)
    pltpu.sync_copy(src_hbm.at[idx_v], rows_v)
    pltpu.sync_copy(rows_v, out_hbm.at[pl.ds(base, FPW)])

  return k(s_wide, fidx)


DCH = 1000


@jax.jit
def _degree(dsts, zrows):
  @functools.partial(
      pl.kernel,
      out_type=jax.ShapeDtypeStruct((2 * NP, HD), f32),
      mesh=_mesh,
      scratch_types=[
          pltpu.VMEM((DCH, HD), f32),
          pltpu.VMEM((DCH,), i32),
          pltpu.VMEM((WQ, HD), f32),
          pltpu.VMEM((WQ,), i32),
          pltpu.VMEM_SHARED((NP, HD), f32),
      ],
      compiler_params=_sc_params,
  )
  def k(dst_hbm, zero_hbm, out_hbm, ones_v, dst_v, stage_v, seq_v, deg_sh):
    c = lax.axis_index("c")
    s = lax.axis_index("s")
    w = _worker_id()
    ones16 = jnp.ones((HD,), f32)

    @pl.loop(0, DCH)
    def _(i):
      ones_v[i] = ones16

    pltpu.sync_copy(zero_hbm.at[pl.ds(s * RPS, RPS)],
                    deg_sh.at[pl.ds(s * RPS, RPS)])
    plsc.subcore_barrier()

    @pl.loop(0, EPD // DCH)
    def _(kk):
      base = w * EPD + kk * DCH
      pltpu.sync_copy(dst_hbm.at[pl.ds(base, DCH)], dst_v)
      pltpu.sync_copy(ones_v, deg_sh.at[dst_v], add=True)

    plsc.subcore_barrier()

    @pl.loop(0, RPS // WQ)
    def _(q):
      rb = s * RPS + q * WQ
      pltpu.sync_copy(deg_sh.at[pl.ds(rb, WQ)], stage_v)

      @pl.loop(0, WQ // HD)
      def _(i):
        seq_v[pl.ds(i * HD, HD)] = (
            lax.iota(i32, HD) + (rb + i * HD)) * 2 + c

      pltpu.sync_copy(stage_v, out_hbm.at[seq_v])

  return k(dsts, zrows)


@jax.jit
def _propagate(z_wide, src2, dsts, zrows):
  @functools.partial(
      pl.kernel,
      out_type=jax.ShapeDtypeStruct((2 * NP, HD), f32),
      mesh=_mesh,
      scratch_types=[
          pltpu.VMEM((CH,), i32),
          pltpu.VMEM((CH,), i32),
          pltpu.VMEM((CH, HD), f32),
          pltpu.VMEM((WQ, HD), f32),
          pltpu.VMEM((WQ,), i32),
          pltpu.VMEM_SHARED((NP, HD), f32),
      ],
      compiler_params=_sc_params,
  )
  def k(z_hbm, src2_hbm, dst_hbm, zero_hbm, out_hbm,
        src_v, dst_v, rows_v, stage_v, seq_v, acc_sh):
    c = lax.axis_index("c")
    s = lax.axis_index("s")
    rbase = s * RPS
    pltpu.sync_copy(zero_hbm.at[pl.ds(rbase, RPS)],
                    acc_sh.at[pl.ds(rbase, RPS)])
    plsc.subcore_barrier()

    def run(src2_c):
      @pl.loop(0, EPS // CH)
      def _(kk):
        base = s * EPS + kk * CH
        pltpu.sync_copy(src2_c.at[pl.ds(base, CH)], src_v)
        pltpu.sync_copy(dst_hbm.at[pl.ds(base, CH)], dst_v)
        pltpu.sync_copy(z_hbm.at[src_v], rows_v)
        pltpu.sync_copy(rows_v, acc_sh.at[dst_v], add=True)

      plsc.subcore_barrier()

      @pl.loop(0, RPS // WQ)
      def _(q):
        rb = rbase + q * WQ
        pltpu.sync_copy(acc_sh.at[pl.ds(rb, WQ)], stage_v)

        @pl.loop(0, WQ // HD)
        def _(i):
          seq_v[pl.ds(i * HD, HD)] = (
              lax.iota(i32, HD) + (rb + i * HD)) * 2 + c

        pltpu.sync_copy(stage_v, out_hbm.at[seq_v])

    @pl.when(c == 0)
    def _():
      run(src2_hbm.at[0])

    @pl.when(c == 1)
    def _():
      run(src2_hbm.at[1])

  return k(z_wide, src2, dsts, zrows)


_BW = 3136


@jax.jit
def _tc_deg_reduce(degw):
  def body(d_ref, dis_ref, inv_ref):
    dblk = d_ref[...]
    half0 = jnp.concatenate([dblk[:, 0:16], dblk[:, 32:48], dblk[:, 64:80],
                             dblk[:, 96:112]], axis=1)
    half1 = jnp.concatenate([dblk[:, 16:32], dblk[:, 48:64], dblk[:, 80:96],
                             dblk[:, 112:128]], axis=1)
    dsum = half0 + half1 + 1.0
    deg = jnp.concatenate(
        [dsum[:, 0:16], dsum[:, 0:16], dsum[:, 16:32], dsum[:, 16:32],
         dsum[:, 32:48], dsum[:, 32:48], dsum[:, 48:64], dsum[:, 48:64]],
        axis=1)
    dis_ref[...] = 1.0 / jnp.sqrt(deg)
    inv_ref[...] = 1.0 / deg

  return pl.pallas_call(
      body,
      out_shape=(
          jax.ShapeDtypeStruct((NW, 128), f32),
          jax.ShapeDtypeStruct((NW, 128), f32),
      ),
      grid=(NW // _BW,),
      in_specs=[pl.BlockSpec((_BW, 128), lambda i: (i, 0))],
      out_specs=(
          pl.BlockSpec((_BW, 128), lambda i: (i, 0)),
          pl.BlockSpec((_BW, 128), lambda i: (i, 0)),
      ),
  )(degw)


@jax.jit
def _tc_pre(x0w, disw, w4):
  def body(x_ref, d_ref, w_ref, xw_ref, z_ref):
    xw = jnp.dot(x_ref[...].astype(jnp.bfloat16), w_ref[...],
                 preferred_element_type=f32)
    xw_ref[...] = xw
    z_ref[...] = xw * d_ref[...]

  return pl.pallas_call(
      body,
      out_shape=(
          jax.ShapeDtypeStruct((NW, 128), f32),
          jax.ShapeDtypeStruct((NW, 128), f32),
      ),
      grid=(NW // _BW,),
      in_specs=[
          pl.BlockSpec((_BW, 128), lambda i: (i, 0)),
          pl.BlockSpec((_BW, 128), lambda i: (i, 0)),
          pl.BlockSpec((128, 128), lambda i: (0, 0)),
      ],
      out_specs=(
          pl.BlockSpec((_BW, 128), lambda i: (i, 0)),
          pl.BlockSpec((_BW, 128), lambda i: (i, 0)),
      ),
  )(x0w, disw, w4)


@jax.jit
def _tc_post(accw, xww, disw, invw, sw, brow, w4n):
  def body(a_ref, xw_ref, d_ref, i_ref, s_ref, b_ref, w_ref,
           so_ref, xwn_ref, zn_ref):
    xl = d_ref[...] * a_ref[...] + i_ref[...] * xw_ref[...] + b_ref[...]
    so_ref[...] = s_ref[...] + xl
    xwn = jnp.dot(xl.astype(jnp.bfloat16), w_ref[...],
                  preferred_element_type=f32)
    xwn_ref[...] = xwn
    zn_ref[...] = xwn * d_ref[...]

  return pl.pallas_call(
      body,
      out_shape=(
          jax.ShapeDtypeStruct((NW, 128), f32),
          jax.ShapeDtypeStruct((NW, 128), f32),
          jax.ShapeDtypeStruct((NW, 128), f32),
      ),
      grid=(NW // _BW,),
      in_specs=[
          pl.BlockSpec((_BW, 128), lambda i: (i, 0)),
          pl.BlockSpec((_BW, 128), lambda i: (i, 0)),
          pl.BlockSpec((_BW, 128), lambda i: (i, 0)),
          pl.BlockSpec((_BW, 128), lambda i: (i, 0)),
          pl.BlockSpec((_BW, 128), lambda i: (i, 0)),
          pl.BlockSpec((1, 128), lambda i: (0, 0)),
          pl.BlockSpec((128, 128), lambda i: (0, 0)),
      ],
      out_specs=(
          pl.BlockSpec((_BW, 128), lambda i: (i, 0)),
          pl.BlockSpec((_BW, 128), lambda i: (i, 0)),
          pl.BlockSpec((_BW, 128), lambda i: (i, 0)),
      ),
  )(accw, xww, disw, invw, sw, brow, w4n)


@jax.jit
def _tc_src2(srcs):
  def body(s_ref, o_ref):
    s2 = s_ref[...] * 2
    o_ref[0] = s2
    o_ref[1] = s2 + 1

  return pl.pallas_call(
      body,
      out_shape=jax.ShapeDtypeStruct((2, EE), i32),
      in_specs=[pl.BlockSpec((EE,), lambda: (0,))],
      out_specs=pl.BlockSpec((2, EE), lambda: (0, 0)),
  )(srcs)


@jax.jit
def _tc_score(rows_w):
  def body(u_ref, v_ref, o_ref):
    p = u_ref[...] * v_ref[...]
    segs = [jnp.sum(p[:, 32 * m:32 * m + 32], axis=1, keepdims=True)
            for m in range(4)]
    o_ref[...] = jnp.concatenate(segs, axis=1) * (1.0 / 16.0)

  nu = 40000
  nbu = nu // 4
  blk = 1000
  return pl.pallas_call(
      body,
      out_shape=jax.ShapeDtypeStruct((nbu, 4), f32),
      grid=(nbu // blk,),
      in_specs=[
          pl.BlockSpec((blk, 128), lambda i: (i, 0)),
          pl.BlockSpec((blk, 128), lambda i: (i + nbu // blk, 0)),
      ],
      out_specs=pl.BlockSpec((blk, 4), lambda i: (i, 0)),
  )(rows_w, rows_w)


def kernel(user_idx, item_idx, item_attr1_idx, item_attr2_idx, edge_index,
           user_table, item_table, attr1_table, attr2_table,
           W1, b1, W2, b2, W3, b3):
  user_idx = user_idx.astype(i32)
  item_idx = item_idx.astype(i32)
  nu = user_idx.shape[0]

  table_cat = jnp.concatenate(
      [user_table, item_table, attr1_table, attr2_table], axis=0)
  idx_all = jnp.concatenate([
      user_idx,
      item_idx + user_table.shape[0],
      item_attr1_idx.astype(i32) + (user_table.shape[0] + item_table.shape[0]),
      item_attr2_idx.astype(i32)
      + (user_table.shape[0] + item_table.shape[0] + attr1_table.shape[0]),
      jnp.zeros((NP - NN,), i32),
  ])
  srcs = edge_index[0].astype(i32)
  dsts = edge_index[1].astype(i32)
  zrows = jnp.zeros((NP, HD), f32)

  def w4_of(W):
    z = jnp.zeros((128, 128), f32)
    for q in range(4):
      z = z.at[32 * q:32 * q + 32, 32 * q:32 * q + 32].set(W)
    return z.astype(jnp.bfloat16)

  w4 = [w4_of(W1), w4_of(W2), w4_of(W3)]
  brows = [jnp.tile(b, 4).reshape(1, 128) for b in (b1, b2, b3)]

  src2 = _tc_src2(srcs)
  x0w = _embed_gather(table_cat, idx_all).reshape(NW, 128)
  degw = _degree(dsts, zrows).reshape(NW, 128)
  disw, invw = _tc_deg_reduce(degw)

  xww, zw = _tc_pre(x0w, disw, w4[0])
  sw = x0w
  for l in range(3):
    accw = _propagate(zw.reshape(2 * NP, HD), src2, dsts,
                      zrows).reshape(NW, 128)
    w4n = w4[l + 1] if l < 2 else w4[2]
    sw, xww, zw = _tc_post(accw, xww, disw, invw, sw, brows[l], w4n)

  fidx = jnp.concatenate(
      [user_idx, item_idx + nu, jnp.zeros((FP - 2 * nu,), i32)])
  rows_w = _final_gather(sw.reshape(NP, D), fidx).reshape(FP // 4, 128)
  score = _tc_score(rows_w)
  return score.reshape(nu)


# speedup vs baseline: 43.9393x; 1.4724x over previous
import functools

import jax
import jax.numpy as jnp
from jax import lax
from jax.experimental import pallas as pl
from jax.experimental.pallas import tpu as pltpu
from jax.experimental.pallas import tpu_sc as plsc

NN = 100000
NP = 100352
EE = 1600000
D = 32
HD = 16
NC, NS = 2, 16
RPW = NP // (NC * NS)
EPD = EE // (NC * NS)
EPS = EE // NS
RPS = NP // NS
CH = 800
WQ = RPS // 8
FP = 80128
FPW = FP // (NC * NS)
NW = NP // 4

_mesh = plsc.VectorSubcoreMesh(core_axis_name="c", subcore_axis_name="s")
_sc_params = pltpu.CompilerParams(use_tc_tiling_on_sc=False)
f32 = jnp.float32
i32 = jnp.int32


def _worker_id():
  return lax.axis_index("s") * NC + lax.axis_index("c")


@jax.jit
def _embed_gather(table_cat, idx_all):
  @functools.partial(
      pl.kernel,
      out_type=jax.ShapeDtypeStruct((NP, D), f32),
      mesh=_mesh,
      scratch_types=[pltpu.VMEM((RPW,), i32), pltpu.VMEM((RPW, D), f32)],
      compiler_params=_sc_params,
  )
  def k(tab_hbm, idx_hbm, out_hbm, idx_v, rows_v):
    base = _worker_id() * RPW
    pltpu.sync_copy(idx_hbm.at[pl.ds(base, RPW)], idx_v)
    pltpu.sync_copy(tab_hbm.at[idx_v], rows_v)
    pltpu.sync_copy(rows_v, out_hbm.at[pl.ds(base, RPW)])

  return k(table_cat, idx_all)


@jax.jit
def _final_gather(s_wide, fidx):
  @functools.partial(
      pl.kernel,
      out_type=jax.ShapeDtypeStruct((FP, D), f32),
      mesh=_mesh,
      scratch_types=[pltpu.VMEM((FPW,), i32), pltpu.VMEM((FPW, D), f32)],
      compiler_params=_sc_params,
  )
  def k(src_hbm, idx_hbm, out_hbm, idx_v, rows_v):
    base = _worker_id() * FPW
    pltpu.sync_copy(idx_hbm.at[pl.ds(base, FPW)], idx_v)
    pltpu.sync_copy(src_hbm.at[idx_v], rows_v)
    pltpu.sync_copy(rows_v, out_hbm.at[pl.ds(base, FPW)])

  return k(s_wide, fidx)


DCH = 1000


@jax.jit
def _degree(dsts, zrows):
  @functools.partial(
      pl.kernel,
      out_type=jax.ShapeDtypeStruct((2 * NP, HD), f32),
      mesh=_mesh,
      scratch_types=[
          pltpu.VMEM((DCH, HD), f32),
          pltpu.VMEM((DCH,), i32),
          pltpu.VMEM((WQ, HD), f32),
          pltpu.VMEM((WQ,), i32),
          pltpu.VMEM_SHARED((NP, HD), f32),
      ],
      compiler_params=_sc_params,
  )
  def k(dst_hbm, zero_hbm, out_hbm, ones_v, dst_v, stage_v, seq_v, deg_sh):
    c = lax.axis_index("c")
    s = lax.axis_index("s")
    w = _worker_id()
    ones16 = jnp.ones((HD,), f32)

    @pl.loop(0, DCH)
    def _(i):
      ones_v[i] = ones16

    pltpu.sync_copy(zero_hbm.at[pl.ds(s * RPS, RPS)],
                    deg_sh.at[pl.ds(s * RPS, RPS)])
    plsc.subcore_barrier()

    @pl.loop(0, EPD // DCH)
    def _(kk):
      base = w * EPD + kk * DCH
      pltpu.sync_copy(dst_hbm.at[pl.ds(base, DCH)], dst_v)
      pltpu.sync_copy(ones_v, deg_sh.at[dst_v], add=True)

    plsc.subcore_barrier()

    @pl.loop(0, RPS // WQ)
    def _(q):
      rb = s * RPS + q * WQ
      pltpu.sync_copy(deg_sh.at[pl.ds(rb, WQ)], stage_v)

      @pl.loop(0, WQ // HD)
      def _(i):
        seq_v[pl.ds(i * HD, HD)] = (
            lax.iota(i32, HD) + (rb + i * HD)) * 2 + c

      pltpu.sync_copy(stage_v, out_hbm.at[seq_v])

  return k(dsts, zrows)


@jax.jit
def _propagate(z_wide, src2, dsts, zrows):
  @functools.partial(
      pl.kernel,
      out_type=jax.ShapeDtypeStruct((2 * NP, HD), f32),
      mesh=_mesh,
      scratch_types=[
          pltpu.VMEM((CH,), i32),
          pltpu.VMEM((CH,), i32),
          pltpu.VMEM((CH,), i32),
          pltpu.VMEM((CH,), i32),
          pltpu.VMEM((CH, HD), f32),
          pltpu.VMEM((CH, HD), f32),
          pltpu.VMEM((WQ,), i32),
          pltpu.VMEM_SHARED((NP, HD), f32),
          pltpu.SemaphoreType.DMA,
          pltpu.SemaphoreType.DMA,
      ],
      compiler_params=_sc_params,
  )
  def k(z_hbm, src2_hbm, dst_hbm, zero_hbm, out_hbm,
        src_v0, src_v1, dst_v0, dst_v1, rows_v0, rows_v1, seq_v, acc_sh,
        sem_g, sem_s):
    c = lax.axis_index("c")
    s = lax.axis_index("s")
    rbase = s * RPS
    pltpu.sync_copy(zero_hbm.at[pl.ds(rbase, RPS)],
                    acc_sh.at[pl.ds(rbase, RPS)])
    plsc.subcore_barrier()

    NCH = EPS // CH
    bufs = ((src_v0, dst_v0, rows_v0), (src_v1, dst_v1, rows_v1))

    def run(src2_c):
      def idx_load(kk, sv, dv):
        base = s * EPS + kk * CH
        pltpu.sync_copy(src2_c.at[pl.ds(base, CH)], sv)
        pltpu.sync_copy(dst_hbm.at[pl.ds(base, CH)], dv)

      def g_start(b):
        pltpu.make_async_copy(z_hbm.at[bufs[b][0]], bufs[b][2], sem_g).start()

      def g_wait(b):
        pltpu.make_async_copy(z_hbm.at[bufs[b][0]], bufs[b][2], sem_g).wait()

      def s_start(b):
        pltpu.make_async_copy(
            bufs[b][2], acc_sh.at[bufs[b][1]], sem_s).start(add=True)

      def s_wait(b):
        pltpu.make_async_copy(
            bufs[b][2], acc_sh.at[bufs[b][1]], sem_s).wait()

      idx_load(0, src_v0, dst_v0)
      g_start(0)
      idx_load(1, src_v1, dst_v1)
      g_wait(0)
      g_start(1)
      s_start(0)

      @pl.loop(0, (NCH - 3) // 2)
      def _(i):
        for b in (0, 1):
          kk = 2 * i + 2 + b
          s_wait(b)
          idx_load(kk, bufs[b][0], bufs[b][1])
          g_wait(1 - b)
          g_start(b)
          s_start(1 - b)

      s_wait(0)
      idx_load(NCH - 1, src_v0, dst_v0)
      g_wait(1)
      g_start(0)
      s_start(1)
      g_wait(0)
      s_wait(1)
      s_start(0)
      s_wait(0)

      plsc.subcore_barrier()

      @pl.loop(0, RPS // WQ)
      def _(q):
        rb = rbase + q * WQ
        pltpu.sync_copy(acc_sh.at[pl.ds(rb, WQ)], rows_v0.at[pl.ds(0, WQ)])

        @pl.loop(0, WQ // HD)
        def _(i):
          seq_v[pl.ds(i * HD, HD)] = (
              lax.iota(i32, HD) + (rb + i * HD)) * 2 + c

        pltpu.sync_copy(rows_v0.at[pl.ds(0, WQ)], out_hbm.at[seq_v])

    @pl.when(c == 0)
    def _():
      run(src2_hbm.at[0])

    @pl.when(c == 1)
    def _():
      run(src2_hbm.at[1])

  return k(z_wide, src2, dsts, zrows)


_BW = 3136


@jax.jit
def _tc_deg_reduce(degw):
  def body(d_ref, dis_ref, inv_ref):
    dblk = d_ref[...]
    half0 = jnp.concatenate([dblk[:, 0:16], dblk[:, 32:48], dblk[:, 64:80],
                             dblk[:, 96:112]], axis=1)
    half1 = jnp.concatenate([dblk[:, 16:32], dblk[:, 48:64], dblk[:, 80:96],
                             dblk[:, 112:128]], axis=1)
    dsum = half0 + half1 + 1.0
    deg = jnp.concatenate(
        [dsum[:, 0:16], dsum[:, 0:16], dsum[:, 16:32], dsum[:, 16:32],
         dsum[:, 32:48], dsum[:, 32:48], dsum[:, 48:64], dsum[:, 48:64]],
        axis=1)
    dis_ref[...] = 1.0 / jnp.sqrt(deg)
    inv_ref[...] = 1.0 / deg

  return pl.pallas_call(
      body,
      out_shape=(
          jax.ShapeDtypeStruct((NW, 128), f32),
          jax.ShapeDtypeStruct((NW, 128), f32),
      ),
      grid=(NW // _BW,),
      in_specs=[pl.BlockSpec((_BW, 128), lambda i: (i, 0))],
      out_specs=(
          pl.BlockSpec((_BW, 128), lambda i: (i, 0)),
          pl.BlockSpec((_BW, 128), lambda i: (i, 0)),
      ),
  )(degw)


@jax.jit
def _tc_pre(x0w, disw, w4):
  def body(x_ref, d_ref, w_ref, xw_ref, z_ref):
    xw = jnp.dot(x_ref[...].astype(jnp.bfloat16), w_ref[...],
                 preferred_element_type=f32)
    xw_ref[...] = xw
    z_ref[...] = xw * d_ref[...]

  return pl.pallas_call(
      body,
      out_shape=(
          jax.ShapeDtypeStruct((NW, 128), f32),
          jax.ShapeDtypeStruct((NW, 128), f32),
      ),
      grid=(NW // _BW,),
      in_specs=[
          pl.BlockSpec((_BW, 128), lambda i: (i, 0)),
          pl.BlockSpec((_BW, 128), lambda i: (i, 0)),
          pl.BlockSpec((128, 128), lambda i: (0, 0)),
      ],
      out_specs=(
          pl.BlockSpec((_BW, 128), lambda i: (i, 0)),
          pl.BlockSpec((_BW, 128), lambda i: (i, 0)),
      ),
  )(x0w, disw, w4)


@jax.jit
def _tc_post(accw, xww, disw, invw, sw, brow, w4n):
  def body(a_ref, xw_ref, d_ref, i_ref, s_ref, b_ref, w_ref,
           so_ref, xwn_ref, zn_ref):
    xl = d_ref[...] * a_ref[...] + i_ref[...] * xw_ref[...] + b_ref[...]
    so_ref[...] = s_ref[...] + xl
    xwn = jnp.dot(xl.astype(jnp.bfloat16), w_ref[...],
                  preferred_element_type=f32)
    xwn_ref[...] = xwn
    zn_ref[...] = xwn * d_ref[...]

  return pl.pallas_call(
      body,
      out_shape=(
          jax.ShapeDtypeStruct((NW, 128), f32),
          jax.ShapeDtypeStruct((NW, 128), f32),
          jax.ShapeDtypeStruct((NW, 128), f32),
      ),
      grid=(NW // _BW,),
      in_specs=[
          pl.BlockSpec((_BW, 128), lambda i: (i, 0)),
          pl.BlockSpec((_BW, 128), lambda i: (i, 0)),
          pl.BlockSpec((_BW, 128), lambda i: (i, 0)),
          pl.BlockSpec((_BW, 128), lambda i: (i, 0)),
          pl.BlockSpec((_BW, 128), lambda i: (i, 0)),
          pl.BlockSpec((1, 128), lambda i: (0, 0)),
          pl.BlockSpec((128, 128), lambda i: (0, 0)),
      ],
      out_specs=(
          pl.BlockSpec((_BW, 128), lambda i: (i, 0)),
          pl.BlockSpec((_BW, 128), lambda i: (i, 0)),
          pl.BlockSpec((_BW, 128), lambda i: (i, 0)),
      ),
  )(accw, xww, disw, invw, sw, brow, w4n)


@jax.jit
def _tc_src2(srcs):
  def body(s_ref, o_ref):
    s2 = s_ref[...] * 2
    o_ref[0] = s2
    o_ref[1] = s2 + 1

  return pl.pallas_call(
      body,
      out_shape=jax.ShapeDtypeStruct((2, EE), i32),
      in_specs=[pl.BlockSpec((EE,), lambda: (0,))],
      out_specs=pl.BlockSpec((2, EE), lambda: (0, 0)),
  )(srcs)


@jax.jit
def _tc_score(rows_w):
  def body(u_ref, v_ref, o_ref):
    p = u_ref[...] * v_ref[...]
    segs = [jnp.sum(p[:, 32 * m:32 * m + 32], axis=1, keepdims=True)
            for m in range(4)]
    o_ref[...] = jnp.concatenate(segs, axis=1) * (1.0 / 16.0)

  nu = 40000
  nbu = nu // 4
  blk = 1000
  return pl.pallas_call(
      body,
      out_shape=jax.ShapeDtypeStruct((nbu, 4), f32),
      grid=(nbu // blk,),
      in_specs=[
          pl.BlockSpec((blk, 128), lambda i: (i, 0)),
          pl.BlockSpec((blk, 128), lambda i: (i + nbu // blk, 0)),
      ],
      out_specs=pl.BlockSpec((blk, 4), lambda i: (i, 0)),
  )(rows_w, rows_w)


def kernel(user_idx, item_idx, item_attr1_idx, item_attr2_idx, edge_index,
           user_table, item_table, attr1_table, attr2_table,
           W1, b1, W2, b2, W3, b3):
  user_idx = user_idx.astype(i32)
  item_idx = item_idx.astype(i32)
  nu = user_idx.shape[0]

  table_cat = jnp.concatenate(
      [user_table, item_table, attr1_table, attr2_table], axis=0)
  idx_all = jnp.concatenate([
      user_idx,
      item_idx + user_table.shape[0],
      item_attr1_idx.astype(i32) + (user_table.shape[0] + item_table.shape[0]),
      item_attr2_idx.astype(i32)
      + (user_table.shape[0] + item_table.shape[0] + attr1_table.shape[0]),
      jnp.zeros((NP - NN,), i32),
  ])
  srcs = edge_index[0].astype(i32)
  dsts = edge_index[1].astype(i32)
  zrows = jnp.zeros((NP, HD), f32)

  eye4 = jnp.eye(4, dtype=f32)

  def w4_of(W):
    return (eye4[:, None, :, None] * W[None, :, None, :]).reshape(
        128, 128).astype(jnp.bfloat16)

  w4 = [w4_of(W1), w4_of(W2), w4_of(W3)]
  brows = [jnp.tile(b, 4).reshape(1, 128) for b in (b1, b2, b3)]

  src2 = _tc_src2(srcs)
  x0w = _embed_gather(table_cat, idx_all).reshape(NW, 128)
  degw = _degree(dsts, zrows).reshape(NW, 128)
  disw, invw = _tc_deg_reduce(degw)

  xww, zw = _tc_pre(x0w, disw, w4[0])
  sw = x0w
  for l in range(3):
    accw = _propagate(zw.reshape(2 * NP, HD), src2, dsts,
                      zrows).reshape(NW, 128)
    w4n = w4[l + 1] if l < 2 else w4[2]
    sw, xww, zw = _tc_post(accw, xww, disw, invw, sw, brows[l], w4n)

  fidx = jnp.concatenate(
      [user_idx, item_idx + nu, jnp.zeros((FP - 2 * nu,), i32)])
  rows_w = _final_gather(sw.reshape(NP, D), fidx).reshape(FP // 4, 128)
  score = _tc_score(rows_w)
  return score.reshape(nu)


# speedup vs baseline: 44.8839x; 1.0215x over previous
import functools

import jax
import jax.numpy as jnp
from jax import lax
from jax.experimental import pallas as pl
from jax.experimental.pallas import tpu as pltpu
from jax.experimental.pallas import tpu_sc as plsc

NN = 100000
NP = 100352
EE = 1600000
D = 32
HD = 16
NC, NS = 2, 16
RPW = NP // (NC * NS)
EPD = EE // (NC * NS)
EPS = EE // NS
RPS = NP // NS
CH = 800
WQ = RPS // 8
FP = 80128
FPW = FP // (NC * NS)
NW = NP // 4

_mesh = plsc.VectorSubcoreMesh(core_axis_name="c", subcore_axis_name="s")
_sc_params = pltpu.CompilerParams(use_tc_tiling_on_sc=False)
f32 = jnp.float32
i32 = jnp.int32


def _worker_id():
  return lax.axis_index("s") * NC + lax.axis_index("c")


@jax.jit
def _embed_gather(table_cat, idx_all):
  @functools.partial(
      pl.kernel,
      out_type=jax.ShapeDtypeStruct((NP, D), f32),
      mesh=_mesh,
      scratch_types=[pltpu.VMEM((RPW,), i32), pltpu.VMEM((RPW, D), f32)],
      compiler_params=_sc_params,
  )
  def k(tab_hbm, idx_hbm, out_hbm, idx_v, rows_v):
    base = _worker_id() * RPW
    pltpu.sync_copy(idx_hbm.at[pl.ds(base, RPW)], idx_v)
    pltpu.sync_copy(tab_hbm.at[idx_v], rows_v)
    pltpu.sync_copy(rows_v, out_hbm.at[pl.ds(base, RPW)])

  return k(table_cat, idx_all)


@jax.jit
def _final_gather(s_wide, fidx):
  @functools.partial(
      pl.kernel,
      out_type=jax.ShapeDtypeStruct((FP, D), f32),
      mesh=_mesh,
      scratch_types=[pltpu.VMEM((FPW,), i32), pltpu.VMEM((FPW, D), f32)],
      compiler_params=_sc_params,
  )
  def k(src_hbm, idx_hbm, out_hbm, idx_v, rows_v):
    base = _worker_id() * FPW
    pltpu.sync_copy(idx_hbm.at[pl.ds(base, FPW)], idx_v)
    pltpu.sync_copy(src_hbm.at[idx_v], rows_v)
    pltpu.sync_copy(rows_v, out_hbm.at[pl.ds(base, FPW)])

  return k(s_wide, fidx)


DCH = 1000


@jax.jit
def _degree(dsts, zrows):
  @functools.partial(
      pl.kernel,
      out_type=jax.ShapeDtypeStruct((2 * NP, HD), f32),
      mesh=_mesh,
      scratch_types=[
          pltpu.VMEM((DCH, HD), f32),
          pltpu.VMEM((DCH,), i32),
          pltpu.VMEM((WQ, HD), f32),
          pltpu.VMEM((WQ,), i32),
          pltpu.VMEM_SHARED((NP, HD), f32),
      ],
      compiler_params=_sc_params,
  )
  def k(dst_hbm, zero_hbm, out_hbm, ones_v, dst_v, stage_v, seq_v, deg_sh):
    c = lax.axis_index("c")
    s = lax.axis_index("s")
    w = _worker_id()
    ones16 = jnp.ones((HD,), f32)

    @pl.loop(0, DCH)
    def _(i):
      ones_v[i] = ones16

    pltpu.sync_copy(zero_hbm.at[pl.ds(s * RPS, RPS)],
                    deg_sh.at[pl.ds(s * RPS, RPS)])
    plsc.subcore_barrier()

    @pl.loop(0, EPD // DCH)
    def _(kk):
      base = w * EPD + kk * DCH
      pltpu.sync_copy(dst_hbm.at[pl.ds(base, DCH)], dst_v)
      pltpu.sync_copy(ones_v, deg_sh.at[dst_v], add=True)

    plsc.subcore_barrier()

    @pl.loop(0, RPS // WQ)
    def _(q):
      rb = s * RPS + q * WQ
      pltpu.sync_copy(deg_sh.at[pl.ds(rb, WQ)], stage_v)

      @pl.loop(0, WQ // HD)
      def _(i):
        seq_v[pl.ds(i * HD, HD)] = (
            lax.iota(i32, HD) + (rb + i * HD)) * 2 + c

      pltpu.sync_copy(stage_v, out_hbm.at[seq_v])

  return k(dsts, zrows)


@jax.jit
def _propagate(z_wide, src2, dsts, zrows):
  @functools.partial(
      pl.kernel,
      out_type=jax.ShapeDtypeStruct((2 * NP, HD), f32),
      mesh=_mesh,
      scratch_types=[
          pltpu.VMEM((CH,), i32),
          pltpu.VMEM((CH,), i32),
          pltpu.VMEM((CH,), i32),
          pltpu.VMEM((CH,), i32),
          pltpu.VMEM((CH, HD), f32),
          pltpu.VMEM((CH, HD), f32),
          pltpu.VMEM((WQ,), i32),
          pltpu.VMEM_SHARED((NP, HD), f32),
          pltpu.SemaphoreType.DMA,
          pltpu.SemaphoreType.DMA,
      ],
      compiler_params=_sc_params,
  )
  def k(z_hbm, src2_hbm, dst_hbm, zero_hbm, out_hbm,
        src_v0, src_v1, dst_v0, dst_v1, rows_v0, rows_v1, seq_v, acc_sh,
        sem_g, sem_s):
    c = lax.axis_index("c")
    s = lax.axis_index("s")
    rbase = s * RPS
    pltpu.sync_copy(zero_hbm.at[pl.ds(rbase, RPS)],
                    acc_sh.at[pl.ds(rbase, RPS)])
    plsc.subcore_barrier()

    NCH = EPS // CH
    bufs = ((src_v0, dst_v0, rows_v0), (src_v1, dst_v1, rows_v1))

    def run(src2_c):
      def idx_load(kk, sv, dv):
        base = s * EPS + kk * CH
        pltpu.sync_copy(src2_c.at[pl.ds(base, CH)], sv)
        pltpu.sync_copy(dst_hbm.at[pl.ds(base, CH)], dv)

      def g_start(b):
        pltpu.make_async_copy(z_hbm.at[bufs[b][0]], bufs[b][2], sem_g).start()

      def g_wait(b):
        pltpu.make_async_copy(z_hbm.at[bufs[b][0]], bufs[b][2], sem_g).wait()

      def s_start(b):
        pltpu.make_async_copy(
            bufs[b][2], acc_sh.at[bufs[b][1]], sem_s).start(add=True)

      def s_wait(b):
        pltpu.make_async_copy(
            bufs[b][2], acc_sh.at[bufs[b][1]], sem_s).wait()

      idx_load(0, src_v0, dst_v0)
      g_start(0)
      idx_load(1, src_v1, dst_v1)
      g_wait(0)
      g_start(1)
      s_start(0)

      @pl.loop(0, (NCH - 3) // 2)
      def _(i):
        for b in (0, 1):
          kk = 2 * i + 2 + b
          s_wait(b)
          idx_load(kk, bufs[b][0], bufs[b][1])
          g_wait(1 - b)
          g_start(b)
          s_start(1 - b)

      s_wait(0)
      idx_load(NCH - 1, src_v0, dst_v0)
      g_wait(1)
      g_start(0)
      s_start(1)
      g_wait(0)
      s_wait(1)
      s_start(0)
      s_wait(0)

      plsc.subcore_barrier()

      @pl.loop(0, RPS // WQ)
      def _(q):
        rb = rbase + q * WQ
        pltpu.sync_copy(acc_sh.at[pl.ds(rb, WQ)], rows_v0.at[pl.ds(0, WQ)])

        @pl.loop(0, WQ // HD)
        def _(i):
          seq_v[pl.ds(i * HD, HD)] = (
              lax.iota(i32, HD) + (rb + i * HD)) * 2 + c

        pltpu.sync_copy(rows_v0.at[pl.ds(0, WQ)], out_hbm.at[seq_v])

    @pl.when(c == 0)
    def _():
      run(src2_hbm.at[0])

    @pl.when(c == 1)
    def _():
      run(src2_hbm.at[1])

  return k(z_wide, src2, dsts, zrows)


_BW = 3136


@jax.jit
def _tc_deg_reduce(degw):
  def body(d_ref, dis_ref):
    dblk = d_ref[...]
    swapped = jnp.concatenate(
        [dblk[:, 16:32], dblk[:, 0:16], dblk[:, 48:64], dblk[:, 32:48],
         dblk[:, 80:96], dblk[:, 64:80], dblk[:, 112:128], dblk[:, 96:112]],
        axis=1)
    deg = dblk + swapped + 1.0
    dis_ref[...] = 1.0 / jnp.sqrt(deg)

  return pl.pallas_call(
      body,
      out_shape=jax.ShapeDtypeStruct((NW, 128), f32),
      grid=(NW // _BW,),
      in_specs=[pl.BlockSpec((_BW, 128), lambda i: (i, 0))],
      out_specs=pl.BlockSpec((_BW, 128), lambda i: (i, 0)),
  )(degw)


@jax.jit
def _tc_pre(x0w, disw, w4):
  def body(x_ref, d_ref, w_ref, xw_ref, z_ref):
    xw = jnp.dot(x_ref[...].astype(jnp.bfloat16), w_ref[...],
                 preferred_element_type=f32)
    xw_ref[...] = xw
    z_ref[...] = xw * d_ref[...]

  return pl.pallas_call(
      body,
      out_shape=(
          jax.ShapeDtypeStruct((NW, 128), f32),
          jax.ShapeDtypeStruct((NW, 128), f32),
      ),
      grid=(NW // _BW,),
      in_specs=[
          pl.BlockSpec((_BW, 128), lambda i: (i, 0)),
          pl.BlockSpec((_BW, 128), lambda i: (i, 0)),
          pl.BlockSpec((128, 128), lambda i: (0, 0)),
      ],
      out_specs=(
          pl.BlockSpec((_BW, 128), lambda i: (i, 0)),
          pl.BlockSpec((_BW, 128), lambda i: (i, 0)),
      ),
  )(x0w, disw, w4)


@jax.jit
def _tc_post(accw, xww, disw, sw, brow, w4n):
  def body(a_ref, xw_ref, d_ref, s_ref, b_ref, w_ref,
           so_ref, xwn_ref, zn_ref):
    dis = d_ref[...]
    xl = dis * a_ref[...] + (dis * dis) * xw_ref[...] + b_ref[...]
    so_ref[...] = s_ref[...] + xl
    xwn = jnp.dot(xl.astype(jnp.bfloat16), w_ref[...],
                  preferred_element_type=f32)
    xwn_ref[...] = xwn
    zn_ref[...] = xwn * dis

  return pl.pallas_call(
      body,
      out_shape=(
          jax.ShapeDtypeStruct((NW, 128), f32),
          jax.ShapeDtypeStruct((NW, 128), f32),
          jax.ShapeDtypeStruct((NW, 128), f32),
      ),
      grid=(NW // _BW,),
      in_specs=[
          pl.BlockSpec((_BW, 128), lambda i: (i, 0)),
          pl.BlockSpec((_BW, 128), lambda i: (i, 0)),
          pl.BlockSpec((_BW, 128), lambda i: (i, 0)),
          pl.BlockSpec((_BW, 128), lambda i: (i, 0)),
          pl.BlockSpec((1, 128), lambda i: (0, 0)),
          pl.BlockSpec((128, 128), lambda i: (0, 0)),
      ],
      out_specs=(
          pl.BlockSpec((_BW, 128), lambda i: (i, 0)),
          pl.BlockSpec((_BW, 128), lambda i: (i, 0)),
          pl.BlockSpec((_BW, 128), lambda i: (i, 0)),
      ),
  )(accw, xww, disw, sw, brow, w4n)


@jax.jit
def _tc_src2(srcs):
  def body(s_ref, o_ref):
    s2 = s_ref[...] * 2
    o_ref[0] = s2
    o_ref[1] = s2 + 1

  return pl.pallas_call(
      body,
      out_shape=jax.ShapeDtypeStruct((2, EE), i32),
      in_specs=[pl.BlockSpec((EE,), lambda: (0,))],
      out_specs=pl.BlockSpec((2, EE), lambda: (0, 0)),
  )(srcs)


@jax.jit
def _tc_score(rows_w):
  def body(u_ref, v_ref, o_ref):
    p = u_ref[...] * v_ref[...]
    segs = [jnp.sum(p[:, 32 * m:32 * m + 32], axis=1, keepdims=True)
            for m in range(4)]
    o_ref[...] = jnp.concatenate(segs, axis=1) * (1.0 / 16.0)

  nu = 40000
  nbu = nu // 4
  blk = 1000
  return pl.pallas_call(
      body,
      out_shape=jax.ShapeDtypeStruct((nbu, 4), f32),
      grid=(nbu // blk,),
      in_specs=[
          pl.BlockSpec((blk, 128), lambda i: (i, 0)),
          pl.BlockSpec((blk, 128), lambda i: (i + nbu // blk, 0)),
      ],
      out_specs=pl.BlockSpec((blk, 4), lambda i: (i, 0)),
  )(rows_w, rows_w)


def kernel(user_idx, item_idx, item_attr1_idx, item_attr2_idx, edge_index,
           user_table, item_table, attr1_table, attr2_table,
           W1, b1, W2, b2, W3, b3):
  user_idx = user_idx.astype(i32)
  item_idx = item_idx.astype(i32)
  nu = user_idx.shape[0]

  table_cat = jnp.concatenate(
      [user_table, item_table, attr1_table, attr2_table], axis=0)
  idx_all = jnp.concatenate([
      user_idx,
      item_idx + user_table.shape[0],
      item_attr1_idx.astype(i32) + (user_table.shape[0] + item_table.shape[0]),
      item_attr2_idx.astype(i32)
      + (user_table.shape[0] + item_table.shape[0] + attr1_table.shape[0]),
      jnp.zeros((NP - NN,), i32),
  ])
  srcs = edge_index[0].astype(i32)
  dsts = edge_index[1].astype(i32)
  zrows = jnp.zeros((NP, HD), f32)

  eye4 = jnp.eye(4, dtype=f32)

  def w4_of(W):
    return (eye4[:, None, :, None] * W[None, :, None, :]).reshape(
        128, 128).astype(jnp.bfloat16)

  w4 = [w4_of(W1), w4_of(W2), w4_of(W3)]
  brows = [jnp.tile(b, 4).reshape(1, 128) for b in (b1, b2, b3)]

  src2 = _tc_src2(srcs)
  x0w = _embed_gather(table_cat, idx_all).reshape(NW, 128)
  degw = _degree(dsts, zrows).reshape(NW, 128)
  disw = _tc_deg_reduce(degw)

  xww, zw = _tc_pre(x0w, disw, w4[0])
  sw = x0w
  for l in range(3):
    accw = _propagate(zw.reshape(2 * NP, HD), src2, dsts,
                      zrows).reshape(NW, 128)
    w4n = w4[l + 1] if l < 2 else w4[2]
    sw, xww, zw = _tc_post(accw, xww, disw, sw, brows[l], w4n)

  fidx = jnp.concatenate(
      [user_idx, item_idx + nu, jnp.zeros((FP - 2 * nu,), i32)])
  rows_w = _final_gather(sw.reshape(NP, D), fidx).reshape(FP // 4, 128)
  score = _tc_score(rows_w)
  return score.reshape(nu)


# speedup vs baseline: 46.6176x; 1.0386x over previous
import functools

import jax
import jax.numpy as jnp
from jax import lax
from jax.experimental import pallas as pl
from jax.experimental.pallas import tpu as pltpu
from jax.experimental.pallas import tpu_sc as plsc

NN = 100000
NP = 100352
EE = 1600000
D = 32
HD = 16
NC, NS = 2, 16
RPW = NP // (NC * NS)
EPD = EE // (NC * NS)
EPS = EE // NS
RPS = NP // NS
CH = 800
WQ = RPS // 8
FP = 80128
FPW = FP // (NC * NS)
NW = NP // 4

_mesh = plsc.VectorSubcoreMesh(core_axis_name="c", subcore_axis_name="s")
_sc_params = pltpu.CompilerParams(use_tc_tiling_on_sc=False)
f32 = jnp.float32
i32 = jnp.int32


def _worker_id():
  return lax.axis_index("s") * NC + lax.axis_index("c")


@jax.jit
def _embed_gather(user_t, item_t, attr_cat, idx_all):
  @functools.partial(
      pl.kernel,
      out_type=jax.ShapeDtypeStruct((NP, D), f32),
      mesh=_mesh,
      scratch_types=[pltpu.VMEM((1280,), i32), pltpu.VMEM((1280, D), f32)],
      compiler_params=_sc_params,
  )
  def k(ut_hbm, it_hbm, at_hbm, idx_hbm, out_hbm, idx_v, rows_v):
    w = _worker_id()

    def gphase(tab, base, n):
      pltpu.sync_copy(idx_hbm.at[pl.ds(base, n)], idx_v.at[pl.ds(0, n)])
      pltpu.sync_copy(tab.at[idx_v.at[pl.ds(0, n)]], rows_v.at[pl.ds(0, n)])
      pltpu.sync_copy(rows_v.at[pl.ds(0, n)], out_hbm.at[pl.ds(base, n)])

    gphase(ut_hbm, w * 1248, 1248)
    gphase(it_hbm, 40000 + w * 1248, 1248)

    @pl.when(w == 0)
    def _():
      gphase(ut_hbm, 39936, 64)
      gphase(it_hbm, 79936, 64)

    @pl.when(w < 16)
    def _():
      gphase(at_hbm, 80000 + w * 1272, 1272)

  return k(user_t, item_t, attr_cat, idx_all)


@jax.jit
def _final_gather(s_wide, fidx):
  @functools.partial(
      pl.kernel,
      out_type=jax.ShapeDtypeStruct((FP, D), f32),
      mesh=_mesh,
      scratch_types=[pltpu.VMEM((FPW,), i32), pltpu.VMEM((FPW, D), f32)],
      compiler_params=_sc_params,
  )
  def k(src_hbm, idx_hbm, out_hbm, idx_v, rows_v):
    base = _worker_id() * FPW
    pltpu.sync_copy(idx_hbm.at[pl.ds(base, FPW)], idx_v)
    pltpu.sync_copy(src_hbm.at[idx_v], rows_v)
    pltpu.sync_copy(rows_v, out_hbm.at[pl.ds(base, FPW)])

  return k(s_wide, fidx)


DCH = 1000


@jax.jit
def _degree(dsts, zrows):
  @functools.partial(
      pl.kernel,
      out_type=jax.ShapeDtypeStruct((2 * NP, HD), f32),
      mesh=_mesh,
      scratch_types=[
          pltpu.VMEM((DCH, HD), f32),
          pltpu.VMEM((DCH,), i32),
          pltpu.VMEM((WQ, HD), f32),
          pltpu.VMEM((WQ,), i32),
          pltpu.VMEM_SHARED((NP, HD), f32),
      ],
      compiler_params=_sc_params,
  )
  def k(dst_hbm, zero_hbm, out_hbm, ones_v, dst_v, stage_v, seq_v, deg_sh):
    c = lax.axis_index("c")
    s = lax.axis_index("s")
    w = _worker_id()
    ones16 = jnp.ones((HD,), f32)

    @pl.loop(0, DCH)
    def _(i):
      ones_v[i] = ones16

    pltpu.sync_copy(zero_hbm.at[pl.ds(s * RPS, RPS)],
                    deg_sh.at[pl.ds(s * RPS, RPS)])
    plsc.subcore_barrier()

    @pl.loop(0, EPD // DCH)
    def _(kk):
      base = w * EPD + kk * DCH
      pltpu.sync_copy(dst_hbm.at[pl.ds(base, DCH)], dst_v)
      pltpu.sync_copy(ones_v, deg_sh.at[dst_v], add=True)

    plsc.subcore_barrier()

    @pl.loop(0, RPS // WQ)
    def _(q):
      rb = s * RPS + q * WQ
      pltpu.sync_copy(deg_sh.at[pl.ds(rb, WQ)], stage_v)

      @pl.loop(0, WQ // HD)
      def _(i):
        seq_v[pl.ds(i * HD, HD)] = (
            lax.iota(i32, HD) + (rb + i * HD)) * 2 + c

      pltpu.sync_copy(stage_v, out_hbm.at[seq_v])

  return k(dsts, zrows)


@jax.jit
def _propagate(z_wide, src2, dsts, zrows):
  @functools.partial(
      pl.kernel,
      out_type=jax.ShapeDtypeStruct((2 * NP, HD), f32),
      mesh=_mesh,
      scratch_types=[
          pltpu.VMEM((CH,), i32),
          pltpu.VMEM((CH,), i32),
          pltpu.VMEM((CH,), i32),
          pltpu.VMEM((CH,), i32),
          pltpu.VMEM((CH, HD), f32),
          pltpu.VMEM((CH, HD), f32),
          pltpu.VMEM((WQ,), i32),
          pltpu.VMEM_SHARED((NP, HD), f32),
          pltpu.SemaphoreType.DMA,
          pltpu.SemaphoreType.DMA,
      ],
      compiler_params=_sc_params,
  )
  def k(z_hbm, src2_hbm, dst_hbm, zero_hbm, out_hbm,
        src_v0, src_v1, dst_v0, dst_v1, rows_v0, rows_v1, seq_v, acc_sh,
        sem_g, sem_s):
    c = lax.axis_index("c")
    s = lax.axis_index("s")
    rbase = s * RPS
    pltpu.sync_copy(zero_hbm.at[pl.ds(rbase, RPS)],
                    acc_sh.at[pl.ds(rbase, RPS)])
    plsc.subcore_barrier()

    NCH = EPS // CH
    bufs = ((src_v0, dst_v0, rows_v0), (src_v1, dst_v1, rows_v1))

    def run(src2_c):
      def idx_load(kk, sv, dv):
        base = s * EPS + kk * CH
        pltpu.sync_copy(src2_c.at[pl.ds(base, CH)], sv)
        pltpu.sync_copy(dst_hbm.at[pl.ds(base, CH)], dv)

      def g_start(b):
        pltpu.make_async_copy(z_hbm.at[bufs[b][0]], bufs[b][2], sem_g).start()

      def g_wait(b):
        pltpu.make_async_copy(z_hbm.at[bufs[b][0]], bufs[b][2], sem_g).wait()

      def s_start(b):
        pltpu.make_async_copy(
            bufs[b][2], acc_sh.at[bufs[b][1]], sem_s).start(add=True)

      def s_wait(b):
        pltpu.make_async_copy(
            bufs[b][2], acc_sh.at[bufs[b][1]], sem_s).wait()

      idx_load(0, src_v0, dst_v0)
      g_start(0)
      idx_load(1, src_v1, dst_v1)
      g_wait(0)
      g_start(1)
      s_start(0)

      @pl.loop(0, (NCH - 3) // 2)
      def _(i):
        for b in (0, 1):
          kk = 2 * i + 2 + b
          s_wait(b)
          idx_load(kk, bufs[b][0], bufs[b][1])
          g_wait(1 - b)
          g_start(b)
          s_start(1 - b)

      s_wait(0)
      idx_load(NCH - 1, src_v0, dst_v0)
      g_wait(1)
      g_start(0)
      s_start(1)
      g_wait(0)
      s_wait(1)
      s_start(0)
      s_wait(0)

      plsc.subcore_barrier()

      @pl.loop(0, RPS // WQ)
      def _(q):
        rb = rbase + q * WQ
        pltpu.sync_copy(acc_sh.at[pl.ds(rb, WQ)], rows_v0.at[pl.ds(0, WQ)])

        @pl.loop(0, WQ // HD)
        def _(i):
          seq_v[pl.ds(i * HD, HD)] = (
              lax.iota(i32, HD) + (rb + i * HD)) * 2 + c

        pltpu.sync_copy(rows_v0.at[pl.ds(0, WQ)], out_hbm.at[seq_v])

    @pl.when(c == 0)
    def _():
      run(src2_hbm.at[0])

    @pl.when(c == 1)
    def _():
      run(src2_hbm.at[1])

  return k(z_wide, src2, dsts, zrows)


_BW = 3136


@jax.jit
def _tc_deg_reduce(degw):
  def body(d_ref, dis_ref):
    dblk = d_ref[...]
    swapped = jnp.concatenate(
        [dblk[:, 16:32], dblk[:, 0:16], dblk[:, 48:64], dblk[:, 32:48],
         dblk[:, 80:96], dblk[:, 64:80], dblk[:, 112:128], dblk[:, 96:112]],
        axis=1)
    deg = dblk + swapped + 1.0
    dis_ref[...] = 1.0 / jnp.sqrt(deg)

  return pl.pallas_call(
      body,
      out_shape=jax.ShapeDtypeStruct((NW, 128), f32),
      grid=(NW // _BW,),
      in_specs=[pl.BlockSpec((_BW, 128), lambda i: (i, 0))],
      out_specs=pl.BlockSpec((_BW, 128), lambda i: (i, 0)),
  )(degw)


@jax.jit
def _tc_pre(x0w, disw, w4):
  def body(x_ref, d_ref, w_ref, xw_ref, z_ref):
    xw = jnp.dot(x_ref[...].astype(jnp.bfloat16), w_ref[...],
                 preferred_element_type=f32)
    xw_ref[...] = xw
    z_ref[...] = xw * d_ref[...]

  return pl.pallas_call(
      body,
      out_shape=(
          jax.ShapeDtypeStruct((NW, 128), f32),
          jax.ShapeDtypeStruct((NW, 128), f32),
      ),
      grid=(NW // _BW,),
      in_specs=[
          pl.BlockSpec((_BW, 128), lambda i: (i, 0)),
          pl.BlockSpec((_BW, 128), lambda i: (i, 0)),
          pl.BlockSpec((128, 128), lambda i: (0, 0)),
      ],
      out_specs=(
          pl.BlockSpec((_BW, 128), lambda i: (i, 0)),
          pl.BlockSpec((_BW, 128), lambda i: (i, 0)),
      ),
  )(x0w, disw, w4)


@jax.jit
def _tc_post(accw, xww, disw, sw, brow, w4n):
  def body(a_ref, xw_ref, d_ref, s_ref, b_ref, w_ref,
           so_ref, xwn_ref, zn_ref):
    dis = d_ref[...]
    xl = dis * a_ref[...] + (dis * dis) * xw_ref[...] + b_ref[...]
    so_ref[...] = s_ref[...] + xl
    xwn = jnp.dot(xl.astype(jnp.bfloat16), w_ref[...],
                  preferred_element_type=f32)
    xwn_ref[...] = xwn
    zn_ref[...] = xwn * dis

  return pl.pallas_call(
      body,
      out_shape=(
          jax.ShapeDtypeStruct((NW, 128), f32),
          jax.ShapeDtypeStruct((NW, 128), f32),
          jax.ShapeDtypeStruct((NW, 128), f32),
      ),
      grid=(NW // _BW,),
      in_specs=[
          pl.BlockSpec((_BW, 128), lambda i: (i, 0)),
          pl.BlockSpec((_BW, 128), lambda i: (i, 0)),
          pl.BlockSpec((_BW, 128), lambda i: (i, 0)),
          pl.BlockSpec((_BW, 128), lambda i: (i, 0)),
          pl.BlockSpec((1, 128), lambda i: (0, 0)),
          pl.BlockSpec((128, 128), lambda i: (0, 0)),
      ],
      out_specs=(
          pl.BlockSpec((_BW, 128), lambda i: (i, 0)),
          pl.BlockSpec((_BW, 128), lambda i: (i, 0)),
          pl.BlockSpec((_BW, 128), lambda i: (i, 0)),
      ),
  )(accw, xww, disw, sw, brow, w4n)


@jax.jit
def _tc_src2(srcs):
  def body(s_ref, o_ref):
    s2 = s_ref[...] * 2
    o_ref[0] = s2
    o_ref[1] = s2 + 1

  return pl.pallas_call(
      body,
      out_shape=jax.ShapeDtypeStruct((2, EE), i32),
      in_specs=[pl.BlockSpec((EE,), lambda: (0,))],
      out_specs=pl.BlockSpec((2, EE), lambda: (0, 0)),
  )(srcs)


@jax.jit
def _tc_score(rows_w):
  def body(u_ref, v_ref, o_ref):
    p = u_ref[...] * v_ref[...]
    segs = [jnp.sum(p[:, 32 * m:32 * m + 32], axis=1, keepdims=True)
            for m in range(4)]
    o_ref[...] = jnp.concatenate(segs, axis=1) * (1.0 / 16.0)

  nu = 40000
  nbu = nu // 4
  blk = 1000
  return pl.pallas_call(
      body,
      out_shape=jax.ShapeDtypeStruct((nbu, 4), f32),
      grid=(nbu // blk,),
      in_specs=[
          pl.BlockSpec((blk, 128), lambda i: (i, 0)),
          pl.BlockSpec((blk, 128), lambda i: (i + nbu // blk, 0)),
      ],
      out_specs=pl.BlockSpec((blk, 4), lambda i: (i, 0)),
  )(rows_w, rows_w)


def kernel(user_idx, item_idx, item_attr1_idx, item_attr2_idx, edge_index,
           user_table, item_table, attr1_table, attr2_table,
           W1, b1, W2, b2, W3, b3):
  user_idx = user_idx.astype(i32)
  item_idx = item_idx.astype(i32)
  nu = user_idx.shape[0]

  attr_cat = jnp.concatenate([attr1_table, attr2_table], axis=0)
  idx_all = jnp.concatenate([
      user_idx,
      item_idx,
      item_attr1_idx.astype(i32),
      item_attr2_idx.astype(i32) + attr1_table.shape[0],
      jnp.zeros((NP - NN,), i32),
  ])
  srcs = edge_index[0].astype(i32)
  dsts = edge_index[1].astype(i32)
  zrows = jnp.zeros((NP, HD), f32)

  eye4 = jnp.eye(4, dtype=f32)

  def w4_of(W):
    return (eye4[:, None, :, None] * W[None, :, None, :]).reshape(
        128, 128).astype(jnp.bfloat16)

  w4 = [w4_of(W1), w4_of(W2), w4_of(W3)]
  brows = [jnp.tile(b, 4).reshape(1, 128) for b in (b1, b2, b3)]

  src2 = _tc_src2(srcs)
  x0w = _embed_gather(user_table, item_table, attr_cat,
                      idx_all).reshape(NW, 128)
  degw = _degree(dsts, zrows).reshape(NW, 128)
  disw = _tc_deg_reduce(degw)

  xww, zw = _tc_pre(x0w, disw, w4[0])
  sw = x0w
  for l in range(3):
    accw = _propagate(zw.reshape(2 * NP, HD), src2, dsts,
                      zrows).reshape(NW, 128)
    w4n = w4[l + 1] if l < 2 else w4[2]
    sw, xww, zw = _tc_post(accw, xww, disw, sw, brows[l], w4n)

  fidx = jnp.concatenate(
      [user_idx, item_idx + nu, jnp.zeros((FP - 2 * nu,), i32)])
  rows_w = _final_gather(sw.reshape(NP, D), fidx).reshape(FP // 4, 128)
  score = _tc_score(rows_w)
  return score.reshape(nu)
